# baseline (speedup 1.0000x reference)
import functools
import math

import jax
import jax.numpy as jnp
from jax.experimental import pallas as pl
from jax.experimental.pallas import tpu as pltpu

_H = 8
_EPS = 1e-5
_CN = 5000


def _ln(x, g, b):
    m = jnp.mean(x, axis=-1, keepdims=True)
    v = jnp.mean((x - m) * (x - m), axis=-1, keepdims=True)
    return (x - m) * jax.lax.rsqrt(v + _EPS) * g + b


def _nt(a, b):
    return jax.lax.dot_general(a, b, (((1,), (1,)), ((), ())),
                               preferred_element_type=jnp.float32)


def _nn(a, b):
    return jax.lax.dot_general(a, b, (((1,), (0,)), ((), ())),
                               preferred_element_type=jnp.float32)


def _row_reduce(x, op, final):
    w = x.shape[1]
    k = w // 128
    if k < 2:
        return final(x)
    parts = [x[:, i * 128:(i + 1) * 128] for i in range(k)]
    while len(parts) > 1:
        nxt = [op(parts[i], parts[i + 1]) for i in range(0, len(parts) - 1, 2)]
        if len(parts) % 2:
            nxt.append(parts[-1])
        parts = nxt
    r = final(parts[0])
    if w % 128:
        r = op(r, final(x[:, k * 128:]))
    return r


def _rowmax(x):
    return _row_reduce(x, jnp.maximum,
                       lambda v: jnp.max(v, axis=-1, keepdims=True))


def _rowsum(x):
    return _row_reduce(x, jnp.add,
                       lambda v: jnp.sum(v, axis=-1, keepdims=True))


def _inst_body(x_ref, ipw_ref, ipb_ref, ipg_ref, ipbb_ref, out_ref):
    pre = _nt(x_ref[0], ipw_ref[...]) + ipb_ref[0]
    out_ref[0] = jax.nn.relu(_ln(pre, ipg_ref[0], ipbb_ref[0])).astype(
        jnp.bfloat16)


def _decoder_body(n_chunks, n_layers, scale,
                  inst_ref, qe_ref,
                  cawqkv_ref, cabqkv_ref, cawo_ref, cabo_ref,
                  sawqkv_ref, sabqkv_ref, sawo_ref, sabo_ref,
                  sag_ref, sab_ref,
                  fw1_ref, fb1_ref, fw2_ref, fb2_ref, fg_ref, fb_ref,
                  og_ref, ob_ref, cw1_ref, cb1_ref, cw2_ref, cb2_ref,
                  qn_out, lab_out,
                  q_state, qp, acc, m_scr, l_scr):
    l = pl.program_id(1)
    c = pl.program_id(2)
    dh = qp.shape[1] // _H

    @pl.when((l == 0) & (c == 0))
    def _():
        q_state[...] = qe_ref[...]

    @pl.when(c == 0)
    def _():
        wq = cawqkv_ref[0, 0:256, :]
        bq = cabqkv_ref[0, 0, 0:256].astype(jnp.float32)
        qp[...] = (_nt(q_state[...].astype(jnp.bfloat16), wq) + bq) * scale
        acc[...] = jnp.zeros_like(acc)
        m_scr[...] = jnp.full_like(m_scr, -1e30)
        l_scr[...] = jnp.zeros_like(l_scr)

    inst = inst_ref[0]
    wk = cawqkv_ref[0, 256:512, :]
    wv = cawqkv_ref[0, 512:768, :]
    kk = _nt(inst, wk).astype(jnp.bfloat16)
    vv = _nt(inst, wv).astype(jnp.bfloat16)
    qp_all = qp[...].astype(jnp.bfloat16)
    m_all = m_scr[...]
    l_all = l_scr[...]
    acc_all = acc[...]
    new_m, new_l, new_acc = [], [], []
    for h in range(_H):
        sl = slice(h * dh, (h + 1) * dh)
        s = _nt(qp_all[:, sl], kk[:, sl])
        m_old = m_all[:, h:h + 1]
        m_new = jnp.maximum(m_old, _rowmax(s))
        p = jnp.exp(s - m_new)
        alpha = jnp.exp(m_old - m_new)
        new_m.append(m_new)
        new_l.append(l_all[:, h:h + 1] * alpha + _rowsum(p))
        new_acc.append(acc_all[:, sl] * alpha +
                       _nn(p.astype(jnp.bfloat16), vv[:, sl]))
    m_scr[...] = jnp.concatenate(new_m, axis=1)
    l_scr[...] = jnp.concatenate(new_l, axis=1)
    acc[...] = jnp.concatenate(new_acc, axis=1)

    @pl.when(c == n_chunks - 1)
    def _():
        q = q_state[...]
        acc_all = acc[...]
        l_all = l_scr[...]
        o_parts = []
        for h in range(_H):
            sl = slice(h * dh, (h + 1) * dh)
            o_parts.append(acc_all[:, sl] * (1.0 / l_all[:, h:h + 1]))
        o = jnp.concatenate(o_parts, axis=1) + cabqkv_ref[0, 0, 512:768]
        q = q + _nt(o, cawo_ref[0]) + cabo_ref[0, 0]

        qkv = _nt(q, sawqkv_ref[0]) + sabqkv_ref[0, 0]
        d = q.shape[1]
        qs, ks, vs = qkv[:, :d], qkv[:, d:2 * d], qkv[:, 2 * d:]
        sa_parts = []
        for h in range(_H):
            sl = slice(h * dh, (h + 1) * dh)
            s = _nt(qs[:, sl], ks[:, sl]) * scale
            s = s - jnp.max(s, axis=-1, keepdims=True)
            p = jnp.exp(s)
            p = p * (1.0 / jnp.sum(p, axis=-1, keepdims=True))
            sa_parts.append(_nn(p, vs[:, sl]))
        sa_o = jnp.concatenate(sa_parts, axis=1)
        q = _ln(q + _nt(sa_o, sawo_ref[0]) + sabo_ref[0, 0],
                sag_ref[0, 0], sab_ref[0, 0])

        h1 = jax.nn.relu(_nt(q, fw1_ref[0]) + fb1_ref[0, 0])
        f2 = _nt(h1, fw2_ref[0]) + fb2_ref[0, 0]
        q = _ln(q + f2, fg_ref[0, 0], fb_ref[0, 0])
        q_state[...] = q

    @pl.when((c == n_chunks - 1) & (l == n_layers - 1))
    def _():
        qn = _ln(q_state[...], og_ref[0], ob_ref[0])
        qn_out[0] = qn
        lh = jax.nn.relu(_nt(qn, cw1_ref[...]) + cb1_ref[0])
        lab_out[0] = _nt(lh, cw2_ref[...]) + cb2_ref[0]


def _mask_body(xm1_ref, xb1_ref, xm2_ref, xb2_ref, qn_ref, x_ref, out_ref):
    xb = x_ref[0]
    h1 = (jax.nn.relu(_nt(xb, xm1_ref[...]) + xb1_ref[0])).astype(jnp.bfloat16)
    mf = (_nt(h1, xm2_ref[...]) + xb2_ref[0]).astype(jnp.bfloat16)
    out_ref[0] = _nt(mf, qn_ref[0].astype(jnp.bfloat16))


@jax.jit
def kernel(x, batch_offsets, ip_w, ip_b, ip_ln_g, ip_ln_b, xm_w1, xm_b1,
           xm_w2, xm_b2, query_embed, ca_wqkv, ca_bqkv, ca_wo, ca_bo,
           sa_wqkv, sa_bqkv, sa_wo, sa_bo, sa_ln_g, sa_ln_b, ffn_w1, ffn_b1,
           ffn_w2, ffn_b2, ffn_ln_g, ffn_ln_b, out_ln_g, out_ln_b,
           cls_w1, cls_b1, cls_w2, cls_b2):
    b = batch_offsets.shape[0] - 1
    n = x.shape[0] // b
    d = ip_w.shape[0]
    d_in = ip_w.shape[1]
    nq = query_embed.shape[0]
    n_layers = ca_wqkv.shape[0]
    hid = ffn_w1.shape[1]
    nc1 = cls_w2.shape[0]
    cn = _CN if n % _CN == 0 else n
    n_chunks = n // cn
    scale = 1.0 / math.sqrt(d // _H)

    x4 = x.reshape(b, n, d_in).astype(jnp.bfloat16)
    ip_w = ip_w.astype(jnp.bfloat16)
    ca_wqkv = ca_wqkv.astype(jnp.bfloat16)
    xm_w1 = xm_w1.astype(jnp.bfloat16)
    xm_w2 = xm_w2.astype(jnp.bfloat16)
    r2 = lambda a: a.reshape(1, -1)
    r3 = lambda a: a.reshape(a.shape[0], 1, a.shape[1])

    full = lambda shp: pl.BlockSpec(shp, lambda bi, li, ci: (0,) * len(shp))
    perl = lambda shp: pl.BlockSpec(shp, lambda bi, li, ci: (li,) + (0,) * (len(shp) - 1))

    inst = pl.pallas_call(
        _inst_body,
        grid=(b, n_chunks),
        in_specs=[
            pl.BlockSpec((1, cn, d_in), lambda bi, ci: (bi, ci, 0)),
            pl.BlockSpec((d, d_in), lambda bi, ci: (0, 0)),
            pl.BlockSpec((1, d), lambda bi, ci: (0, 0)),
            pl.BlockSpec((1, d), lambda bi, ci: (0, 0)),
            pl.BlockSpec((1, d), lambda bi, ci: (0, 0)),
        ],
        out_specs=pl.BlockSpec((1, cn, d), lambda bi, ci: (bi, ci, 0)),
        out_shape=jax.ShapeDtypeStruct((b, n, d), jnp.bfloat16),
        compiler_params=pltpu.CompilerParams(
            dimension_semantics=("parallel", "arbitrary"),
            vmem_limit_bytes=48 * 1024 * 1024,
        ),
        name="spinst",
    )(x4, ip_w, r2(ip_b), r2(ip_ln_g), r2(ip_ln_b))

    qn_res, labels = pl.pallas_call(
        functools.partial(_decoder_body, n_chunks, n_layers, scale),
        grid=(b, n_layers, n_chunks),
        in_specs=[
            pl.BlockSpec((1, cn, d), lambda bi, li, ci: (bi, ci, 0)),
            full((nq, d)),
            perl((1, 3 * d, d)), perl((1, 1, 3 * d)),
            perl((1, d, d)), perl((1, 1, d)),
            perl((1, 3 * d, d)), perl((1, 1, 3 * d)),
            perl((1, d, d)), perl((1, 1, d)),
            perl((1, 1, d)), perl((1, 1, d)),
            perl((1, hid, d)), perl((1, 1, hid)),
            perl((1, d, hid)), perl((1, 1, d)),
            perl((1, 1, d)), perl((1, 1, d)),
            full((1, d)), full((1, d)),
            full((d, d)), full((1, d)),
            full((nc1, d)), full((1, nc1)),
        ],
        out_specs=[
            pl.BlockSpec((1, nq, d), lambda bi, li, ci: (bi, 0, 0)),
            pl.BlockSpec((1, nq, nc1), lambda bi, li, ci: (bi, 0, 0)),
        ],
        out_shape=[
            jax.ShapeDtypeStruct((b, nq, d), jnp.float32),
            jax.ShapeDtypeStruct((b, nq, nc1), jnp.float32),
        ],
        scratch_shapes=[
            pltpu.VMEM((nq, d), jnp.float32),
            pltpu.VMEM((nq, d), jnp.float32),
            pltpu.VMEM((nq, d), jnp.float32),
            pltpu.VMEM((nq, _H), jnp.float32),
            pltpu.VMEM((nq, _H), jnp.float32),
        ],
        compiler_params=pltpu.CompilerParams(
            dimension_semantics=("parallel", "arbitrary", "arbitrary"),
            vmem_limit_bytes=48 * 1024 * 1024,
        ),
        name="spdecoder",
    )(inst, query_embed,
      ca_wqkv, r3(ca_bqkv), ca_wo, r3(ca_bo),
      sa_wqkv, r3(sa_bqkv), sa_wo, r3(sa_bo), r3(sa_ln_g), r3(sa_ln_b),
      ffn_w1, r3(ffn_b1), ffn_w2, r3(ffn_b2), r3(ffn_ln_g), r3(ffn_ln_b),
      r2(out_ln_g), r2(out_ln_b), cls_w1, r2(cls_b1), cls_w2, r2(cls_b2))

    masks_t = pl.pallas_call(
        _mask_body,
        grid=(b, n_chunks),
        in_specs=[
            pl.BlockSpec((d, d_in), lambda bi, ci: (0, 0)),
            pl.BlockSpec((1, d), lambda bi, ci: (0, 0)),
            pl.BlockSpec((d, d), lambda bi, ci: (0, 0)),
            pl.BlockSpec((1, d), lambda bi, ci: (0, 0)),
            pl.BlockSpec((1, nq, d), lambda bi, ci: (bi, 0, 0)),
            pl.BlockSpec((1, cn, d_in), lambda bi, ci: (bi, ci, 0)),
        ],
        out_specs=pl.BlockSpec((1, cn, nq), lambda bi, ci: (bi, ci, 0)),
        out_shape=jax.ShapeDtypeStruct((b, n, nq), jnp.float32),
        compiler_params=pltpu.CompilerParams(
            dimension_semantics=("parallel", "arbitrary"),
            vmem_limit_bytes=48 * 1024 * 1024,
        ),
        name="spmasks",
    )(xm_w1, r2(xm_b1), xm_w2, r2(xm_b2), qn_res, x4)

    return labels, masks_t.transpose(0, 2, 1)


# speedup vs baseline: 1.0577x; 1.0577x over previous
import functools
import math

import jax
import jax.numpy as jnp
from jax.experimental import pallas as pl
from jax.experimental.pallas import tpu as pltpu

_H = 8
_EPS = 1e-5
_CN = 5000


def _ln(x, g, b):
    m = jnp.mean(x, axis=-1, keepdims=True)
    v = jnp.mean((x - m) * (x - m), axis=-1, keepdims=True)
    return (x - m) * jax.lax.rsqrt(v + _EPS) * g + b


def _nt(a, b):
    return jax.lax.dot_general(a, b, (((1,), (1,)), ((), ())),
                               preferred_element_type=jnp.float32)


def _nn(a, b):
    return jax.lax.dot_general(a, b, (((1,), (0,)), ((), ())),
                               preferred_element_type=jnp.float32)


def _inst_body(x_ref, ipw_ref, ipb_ref, ipg_ref, ipbb_ref, out_ref):
    pre = _nt(x_ref[0], ipw_ref[...]) + ipb_ref[0]
    out_ref[0] = jax.nn.relu(_ln(pre, ipg_ref[0], ipbb_ref[0])).astype(
        jnp.bfloat16)


def _decoder_body(n_chunks, n_layers, scale,
                  inst_ref, qe_ref,
                  cawqkv_ref, cabqkv_ref, cawo_ref, cabo_ref,
                  sawqkv_ref, sabqkv_ref, sawo_ref, sabo_ref,
                  sag_ref, sab_ref,
                  fw1_ref, fb1_ref, fw2_ref, fb2_ref, fg_ref, fb_ref,
                  og_ref, ob_ref, cw1_ref, cb1_ref, cw2_ref, cb2_ref,
                  qn_out, lab_out,
                  q_state, qp, acc, m_scr, l_scr):
    l = pl.program_id(1)
    c = pl.program_id(2)
    dh = qp.shape[1] // _H

    @pl.when((l == 0) & (c == 0))
    def _():
        q_state[...] = qe_ref[...]

    @pl.when(c == 0)
    def _():
        wq = cawqkv_ref[0, 0:256, :]
        bq = cabqkv_ref[0, 0, 0:256].astype(jnp.float32)
        qp[...] = (_nt(q_state[...].astype(jnp.bfloat16), wq) + bq) * scale
        acc[...] = jnp.zeros_like(acc)
        m_scr[...] = jnp.full_like(m_scr, -1e30)
        l_scr[...] = jnp.zeros_like(l_scr)

    inst = inst_ref[0]
    wk = cawqkv_ref[0, 256:512, :]
    wv = cawqkv_ref[0, 512:768, :]
    kk_t = _nt(wk, inst).astype(jnp.bfloat16)
    vv = _nt(inst, wv).astype(jnp.bfloat16)
    qp_all = qp[...].astype(jnp.bfloat16)
    m_all = m_scr[...]
    l_all = l_scr[...]
    acc_all = acc[...]
    new_m, new_l, new_acc = [], [], []
    for h in range(_H):
        sl = slice(h * dh, (h + 1) * dh)
        s = _nn(qp_all[:, sl], kk_t[sl, :])
        m_old = m_all[:, h:h + 1]
        m_new = jnp.maximum(m_old, jnp.max(s, axis=-1, keepdims=True))
        p = jnp.exp(s - m_new)
        alpha = jnp.exp(m_old - m_new)
        new_m.append(m_new)
        new_l.append(l_all[:, h:h + 1] * alpha + jnp.sum(p, axis=-1, keepdims=True))
        new_acc.append(acc_all[:, sl] * alpha +
                       _nn(p.astype(jnp.bfloat16), vv)[:, sl])
    m_scr[...] = jnp.concatenate(new_m, axis=1)
    l_scr[...] = jnp.concatenate(new_l, axis=1)
    acc[...] = jnp.concatenate(new_acc, axis=1)

    @pl.when(c == n_chunks - 1)
    def _():
        q = q_state[...]
        acc_all = acc[...]
        l_all = l_scr[...]
        o_parts = []
        for h in range(_H):
            sl = slice(h * dh, (h + 1) * dh)
            o_parts.append(acc_all[:, sl] * (1.0 / l_all[:, h:h + 1]))
        o = jnp.concatenate(o_parts, axis=1) + cabqkv_ref[0, 0, 512:768]
        q = q + _nt(o, cawo_ref[0]) + cabo_ref[0, 0]

        qkv = _nt(q, sawqkv_ref[0]) + sabqkv_ref[0, 0]
        d = q.shape[1]
        qs, ks, vs = qkv[:, :d], qkv[:, d:2 * d], qkv[:, 2 * d:]
        sa_parts = []
        for h in range(_H):
            sl = slice(h * dh, (h + 1) * dh)
            s = _nt(qs[:, sl], ks[:, sl]) * scale
            s = s - jnp.max(s, axis=-1, keepdims=True)
            p = jnp.exp(s)
            p = p * (1.0 / jnp.sum(p, axis=-1, keepdims=True))
            sa_parts.append(_nn(p, vs[:, sl]))
        sa_o = jnp.concatenate(sa_parts, axis=1)
        q = _ln(q + _nt(sa_o, sawo_ref[0]) + sabo_ref[0, 0],
                sag_ref[0, 0], sab_ref[0, 0])

        h1 = jax.nn.relu(_nt(q, fw1_ref[0]) + fb1_ref[0, 0])
        f2 = _nt(h1, fw2_ref[0]) + fb2_ref[0, 0]
        q = _ln(q + f2, fg_ref[0, 0], fb_ref[0, 0])
        q_state[...] = q

    @pl.when((c == n_chunks - 1) & (l == n_layers - 1))
    def _():
        qn = _ln(q_state[...], og_ref[0], ob_ref[0])
        qn_out[0] = qn
        lh = jax.nn.relu(_nt(qn, cw1_ref[...]) + cb1_ref[0])
        lab_out[0] = _nt(lh, cw2_ref[...]) + cb2_ref[0]


def _mask_body(xm1_ref, xb1_ref, xm2_ref, xb2_ref, qn_ref, x_ref, out_ref):
    xb = x_ref[0]
    h1 = (jax.nn.relu(_nt(xb, xm1_ref[...]) + xb1_ref[0])).astype(jnp.bfloat16)
    mf = (_nt(h1, xm2_ref[...]) + xb2_ref[0]).astype(jnp.bfloat16)
    out_ref[0] = _nt(mf, qn_ref[0].astype(jnp.bfloat16))


@jax.jit
def kernel(x, batch_offsets, ip_w, ip_b, ip_ln_g, ip_ln_b, xm_w1, xm_b1,
           xm_w2, xm_b2, query_embed, ca_wqkv, ca_bqkv, ca_wo, ca_bo,
           sa_wqkv, sa_bqkv, sa_wo, sa_bo, sa_ln_g, sa_ln_b, ffn_w1, ffn_b1,
           ffn_w2, ffn_b2, ffn_ln_g, ffn_ln_b, out_ln_g, out_ln_b,
           cls_w1, cls_b1, cls_w2, cls_b2):
    b = batch_offsets.shape[0] - 1
    n = x.shape[0] // b
    d = ip_w.shape[0]
    d_in = ip_w.shape[1]
    nq = query_embed.shape[0]
    n_layers = ca_wqkv.shape[0]
    hid = ffn_w1.shape[1]
    nc1 = cls_w2.shape[0]
    cn = _CN if n % _CN == 0 else n
    n_chunks = n // cn
    scale = 1.0 / math.sqrt(d // _H)

    x4 = x.reshape(b, n, d_in).astype(jnp.bfloat16)
    ip_w = ip_w.astype(jnp.bfloat16)
    ca_wqkv = ca_wqkv.astype(jnp.bfloat16)
    xm_w1 = xm_w1.astype(jnp.bfloat16)
    xm_w2 = xm_w2.astype(jnp.bfloat16)
    r2 = lambda a: a.reshape(1, -1)
    r3 = lambda a: a.reshape(a.shape[0], 1, a.shape[1])

    full = lambda shp: pl.BlockSpec(shp, lambda bi, li, ci: (0,) * len(shp))
    perl = lambda shp: pl.BlockSpec(shp, lambda bi, li, ci: (li,) + (0,) * (len(shp) - 1))

    inst = pl.pallas_call(
        _inst_body,
        grid=(b, n_chunks),
        in_specs=[
            pl.BlockSpec((1, cn, d_in), lambda bi, ci: (bi, ci, 0)),
            pl.BlockSpec((d, d_in), lambda bi, ci: (0, 0)),
            pl.BlockSpec((1, d), lambda bi, ci: (0, 0)),
            pl.BlockSpec((1, d), lambda bi, ci: (0, 0)),
            pl.BlockSpec((1, d), lambda bi, ci: (0, 0)),
        ],
        out_specs=pl.BlockSpec((1, cn, d), lambda bi, ci: (bi, ci, 0)),
        out_shape=jax.ShapeDtypeStruct((b, n, d), jnp.bfloat16),
        compiler_params=pltpu.CompilerParams(
            dimension_semantics=("parallel", "arbitrary"),
            vmem_limit_bytes=48 * 1024 * 1024,
        ),
        name="spinst",
    )(x4, ip_w, r2(ip_b), r2(ip_ln_g), r2(ip_ln_b))

    qn_res, labels = pl.pallas_call(
        functools.partial(_decoder_body, n_chunks, n_layers, scale),
        grid=(b, n_layers, n_chunks),
        in_specs=[
            pl.BlockSpec((1, cn, d), lambda bi, li, ci: (bi, ci, 0)),
            full((nq, d)),
            perl((1, 3 * d, d)), perl((1, 1, 3 * d)),
            perl((1, d, d)), perl((1, 1, d)),
            perl((1, 3 * d, d)), perl((1, 1, 3 * d)),
            perl((1, d, d)), perl((1, 1, d)),
            perl((1, 1, d)), perl((1, 1, d)),
            perl((1, hid, d)), perl((1, 1, hid)),
            perl((1, d, hid)), perl((1, 1, d)),
            perl((1, 1, d)), perl((1, 1, d)),
            full((1, d)), full((1, d)),
            full((d, d)), full((1, d)),
            full((nc1, d)), full((1, nc1)),
        ],
        out_specs=[
            pl.BlockSpec((1, nq, d), lambda bi, li, ci: (bi, 0, 0)),
            pl.BlockSpec((1, nq, nc1), lambda bi, li, ci: (bi, 0, 0)),
        ],
        out_shape=[
            jax.ShapeDtypeStruct((b, nq, d), jnp.float32),
            jax.ShapeDtypeStruct((b, nq, nc1), jnp.float32),
        ],
        scratch_shapes=[
            pltpu.VMEM((nq, d), jnp.float32),
            pltpu.VMEM((nq, d), jnp.float32),
            pltpu.VMEM((nq, d), jnp.float32),
            pltpu.VMEM((nq, _H), jnp.float32),
            pltpu.VMEM((nq, _H), jnp.float32),
        ],
        compiler_params=pltpu.CompilerParams(
            dimension_semantics=("parallel", "arbitrary", "arbitrary"),
            vmem_limit_bytes=48 * 1024 * 1024,
        ),
        name="spdecoder",
    )(inst, query_embed,
      ca_wqkv, r3(ca_bqkv), ca_wo, r3(ca_bo),
      sa_wqkv, r3(sa_bqkv), sa_wo, r3(sa_bo), r3(sa_ln_g), r3(sa_ln_b),
      ffn_w1, r3(ffn_b1), ffn_w2, r3(ffn_b2), r3(ffn_ln_g), r3(ffn_ln_b),
      r2(out_ln_g), r2(out_ln_b), cls_w1, r2(cls_b1), cls_w2, r2(cls_b2))

    masks_t = pl.pallas_call(
        _mask_body,
        grid=(b, n_chunks),
        in_specs=[
            pl.BlockSpec((d, d_in), lambda bi, ci: (0, 0)),
            pl.BlockSpec((1, d), lambda bi, ci: (0, 0)),
            pl.BlockSpec((d, d), lambda bi, ci: (0, 0)),
            pl.BlockSpec((1, d), lambda bi, ci: (0, 0)),
            pl.BlockSpec((1, nq, d), lambda bi, ci: (bi, 0, 0)),
            pl.BlockSpec((1, cn, d_in), lambda bi, ci: (bi, ci, 0)),
        ],
        out_specs=pl.BlockSpec((1, cn, nq), lambda bi, ci: (bi, ci, 0)),
        out_shape=jax.ShapeDtypeStruct((b, n, nq), jnp.float32),
        compiler_params=pltpu.CompilerParams(
            dimension_semantics=("parallel", "arbitrary"),
            vmem_limit_bytes=48 * 1024 * 1024,
        ),
        name="spmasks",
    )(xm_w1, r2(xm_b1), xm_w2, r2(xm_b2), qn_res, x4)

    return labels, masks_t.transpose(0, 2, 1)


# speedup vs baseline: 1.0892x; 1.0297x over previous
import functools
import math

import jax
import jax.numpy as jnp
from jax.experimental import pallas as pl
from jax.experimental.pallas import tpu as pltpu

_H = 8
_EPS = 1e-5
_CN = 5000


def _ln(x, g, b):
    m = jnp.mean(x, axis=-1, keepdims=True)
    v = jnp.mean((x - m) * (x - m), axis=-1, keepdims=True)
    return (x - m) * jax.lax.rsqrt(v + _EPS) * g + b


def _nt(a, b):
    return jax.lax.dot_general(a, b, (((1,), (1,)), ((), ())),
                               preferred_element_type=jnp.float32)


def _nn(a, b):
    return jax.lax.dot_general(a, b, (((1,), (0,)), ((), ())),
                               preferred_element_type=jnp.float32)


def _inst_body(x_ref, ipw_ref, ipb_ref, ipg_ref, ipbb_ref, out_ref):
    pre = _nt(x_ref[0], ipw_ref[...]) + ipb_ref[0]
    out_ref[0] = jax.nn.relu(_ln(pre, ipg_ref[0], ipbb_ref[0])).astype(
        jnp.bfloat16)


def _decoder_body(n_chunks, n_layers, scale,
                  inst_ref, qe_ref,
                  cawqkv_ref, cabqkv_ref, cawo_ref, cabo_ref,
                  sawqkv_ref, sabqkv_ref, sawo_ref, sabo_ref,
                  sag_ref, sab_ref,
                  fw1_ref, fb1_ref, fw2_ref, fb2_ref, fg_ref, fb_ref,
                  og_ref, ob_ref, cw1_ref, cb1_ref, cw2_ref, cb2_ref,
                  qn_out, lab_out,
                  q_state, qp, acc, m_scr, l_scr):
    l = pl.program_id(1)
    c = pl.program_id(2)
    dh = qp.shape[2] // _H
    n_grp = qp.shape[0]

    @pl.when((l == 0) & (c == 0))
    def _():
        for g in range(n_grp):
            q_state[g] = qe_ref[...]

    @pl.when(c == 0)
    def _():
        wq = cawqkv_ref[0, 0:256, :]
        bq = cabqkv_ref[0, 0, 0:256].astype(jnp.float32)
        for g in range(n_grp):
            qp[g] = (_nt(q_state[g].astype(jnp.bfloat16), wq) + bq) * scale
        acc[...] = jnp.zeros_like(acc)
        m_scr[...] = jnp.full_like(m_scr, -1e30)
        l_scr[...] = jnp.zeros_like(l_scr)

    wk = cawqkv_ref[0, 256:512, :]
    wv = cawqkv_ref[0, 512:768, :]
    for g in range(n_grp):
        inst = inst_ref[0, g, 0]
        kk_t = _nt(wk, inst).astype(jnp.bfloat16)
        vv = _nt(inst, wv).astype(jnp.bfloat16)
        qp_all = qp[g].astype(jnp.bfloat16)
        m_all = m_scr[g]
        l_all = l_scr[g]
        acc_all = acc[g]
        new_m, new_l, new_acc = [], [], []
        for h in range(_H):
            sl = slice(h * dh, (h + 1) * dh)
            s = _nn(qp_all[:, sl], kk_t[sl, :])
            m_old = m_all[:, h:h + 1]
            m_new = jnp.maximum(m_old, jnp.max(s, axis=-1, keepdims=True))
            p = jnp.exp(s - m_new)
            alpha = jnp.exp(m_old - m_new)
            new_m.append(m_new)
            new_l.append(l_all[:, h:h + 1] * alpha +
                         jnp.sum(p, axis=-1, keepdims=True))
            new_acc.append(acc_all[:, sl] * alpha +
                           _nn(p.astype(jnp.bfloat16), vv)[:, sl])
        m_scr[g] = jnp.concatenate(new_m, axis=1)
        l_scr[g] = jnp.concatenate(new_l, axis=1)
        acc[g] = jnp.concatenate(new_acc, axis=1)

    @pl.when(c == n_chunks - 1)
    def _():
        for g in range(n_grp):
            q = q_state[g]
            acc_all = acc[g]
            l_all = l_scr[g]
            o_parts = []
            for h in range(_H):
                sl = slice(h * dh, (h + 1) * dh)
                o_parts.append(acc_all[:, sl] * (1.0 / l_all[:, h:h + 1]))
            o = jnp.concatenate(o_parts, axis=1) + cabqkv_ref[0, 0, 512:768]
            q = q + _nt(o, cawo_ref[0]) + cabo_ref[0, 0]

            qkv = _nt(q, sawqkv_ref[0]) + sabqkv_ref[0, 0]
            d = q.shape[1]
            qs, ks, vs = qkv[:, :d], qkv[:, d:2 * d], qkv[:, 2 * d:]
            sa_parts = []
            for h in range(_H):
                sl = slice(h * dh, (h + 1) * dh)
                s = _nt(qs[:, sl], ks[:, sl]) * scale
                s = s - jnp.max(s, axis=-1, keepdims=True)
                p = jnp.exp(s)
                p = p * (1.0 / jnp.sum(p, axis=-1, keepdims=True))
                sa_parts.append(_nn(p, vs[:, sl]))
            sa_o = jnp.concatenate(sa_parts, axis=1)
            q = _ln(q + _nt(sa_o, sawo_ref[0]) + sabo_ref[0, 0],
                    sag_ref[0, 0], sab_ref[0, 0])

            h1 = jax.nn.relu(_nt(q, fw1_ref[0]) + fb1_ref[0, 0])
            f2 = _nt(h1, fw2_ref[0]) + fb2_ref[0, 0]
            q = _ln(q + f2, fg_ref[0, 0], fb_ref[0, 0])
            q_state[g] = q

    @pl.when((c == n_chunks - 1) & (l == n_layers - 1))
    def _():
        for g in range(n_grp):
            qn = _ln(q_state[g], og_ref[0], ob_ref[0])
            qn_out[0, g] = qn
            lh = jax.nn.relu(_nt(qn, cw1_ref[...]) + cb1_ref[0])
            lab_out[0, g] = _nt(lh, cw2_ref[...]) + cb2_ref[0]


def _mask_body(xm1_ref, xb1_ref, xm2_ref, xb2_ref, qn_ref, x_ref, out_ref):
    xb = x_ref[0]
    h1 = (jax.nn.relu(_nt(xb, xm1_ref[...]) + xb1_ref[0])).astype(jnp.bfloat16)
    mf = (_nt(h1, xm2_ref[...]) + xb2_ref[0]).astype(jnp.bfloat16)
    out_ref[0] = _nt(mf, qn_ref[0].astype(jnp.bfloat16))


@jax.jit
def kernel(x, batch_offsets, ip_w, ip_b, ip_ln_g, ip_ln_b, xm_w1, xm_b1,
           xm_w2, xm_b2, query_embed, ca_wqkv, ca_bqkv, ca_wo, ca_bo,
           sa_wqkv, sa_bqkv, sa_wo, sa_bo, sa_ln_g, sa_ln_b, ffn_w1, ffn_b1,
           ffn_w2, ffn_b2, ffn_ln_g, ffn_ln_b, out_ln_g, out_ln_b,
           cls_w1, cls_b1, cls_w2, cls_b2):
    b = batch_offsets.shape[0] - 1
    n = x.shape[0] // b
    d = ip_w.shape[0]
    d_in = ip_w.shape[1]
    nq = query_embed.shape[0]
    n_layers = ca_wqkv.shape[0]
    hid = ffn_w1.shape[1]
    nc1 = cls_w2.shape[0]
    cn = _CN if n % _CN == 0 else n
    n_chunks = n // cn
    scale = 1.0 / math.sqrt(d // _H)

    x4 = x.reshape(b, n, d_in).astype(jnp.bfloat16)
    ip_w = ip_w.astype(jnp.bfloat16)
    ca_wqkv = ca_wqkv.astype(jnp.bfloat16)
    xm_w1 = xm_w1.astype(jnp.bfloat16)
    xm_w2 = xm_w2.astype(jnp.bfloat16)
    r2 = lambda a: a.reshape(1, -1)
    r3 = lambda a: a.reshape(a.shape[0], 1, a.shape[1])

    full = lambda shp: pl.BlockSpec(shp, lambda bi, li, ci: (0,) * len(shp))
    perl = lambda shp: pl.BlockSpec(shp, lambda bi, li, ci: (li,) + (0,) * (len(shp) - 1))

    inst = pl.pallas_call(
        _inst_body,
        grid=(b, n_chunks),
        in_specs=[
            pl.BlockSpec((1, cn, d_in), lambda bi, ci: (bi, ci, 0)),
            pl.BlockSpec((d, d_in), lambda bi, ci: (0, 0)),
            pl.BlockSpec((1, d), lambda bi, ci: (0, 0)),
            pl.BlockSpec((1, d), lambda bi, ci: (0, 0)),
            pl.BlockSpec((1, d), lambda bi, ci: (0, 0)),
        ],
        out_specs=pl.BlockSpec((1, cn, d), lambda bi, ci: (bi, ci, 0)),
        out_shape=jax.ShapeDtypeStruct((b, n, d), jnp.bfloat16),
        compiler_params=pltpu.CompilerParams(
            dimension_semantics=("parallel", "arbitrary"),
            vmem_limit_bytes=48 * 1024 * 1024,
        ),
        name="spinst",
    )(x4, ip_w, r2(ip_b), r2(ip_ln_g), r2(ip_ln_b))

    gsz = 2 if b % 2 == 0 else 1
    n_grp = b // gsz
    inst5 = inst.reshape(n_grp, gsz, n_chunks, cn, d)

    qn_res, labels = pl.pallas_call(
        functools.partial(_decoder_body, n_chunks, n_layers, scale),
        grid=(n_grp, n_layers, n_chunks),
        in_specs=[
            pl.BlockSpec((1, gsz, 1, cn, d), lambda bi, li, ci: (bi, 0, ci, 0, 0)),
            full((nq, d)),
            perl((1, 3 * d, d)), perl((1, 1, 3 * d)),
            perl((1, d, d)), perl((1, 1, d)),
            perl((1, 3 * d, d)), perl((1, 1, 3 * d)),
            perl((1, d, d)), perl((1, 1, d)),
            perl((1, 1, d)), perl((1, 1, d)),
            perl((1, hid, d)), perl((1, 1, hid)),
            perl((1, d, hid)), perl((1, 1, d)),
            perl((1, 1, d)), perl((1, 1, d)),
            full((1, d)), full((1, d)),
            full((d, d)), full((1, d)),
            full((nc1, d)), full((1, nc1)),
        ],
        out_specs=[
            pl.BlockSpec((1, gsz, nq, d), lambda bi, li, ci: (bi, 0, 0, 0)),
            pl.BlockSpec((1, gsz, nq, nc1), lambda bi, li, ci: (bi, 0, 0, 0)),
        ],
        out_shape=[
            jax.ShapeDtypeStruct((n_grp, gsz, nq, d), jnp.float32),
            jax.ShapeDtypeStruct((n_grp, gsz, nq, nc1), jnp.float32),
        ],
        scratch_shapes=[
            pltpu.VMEM((gsz, nq, d), jnp.float32),
            pltpu.VMEM((gsz, nq, d), jnp.float32),
            pltpu.VMEM((gsz, nq, d), jnp.float32),
            pltpu.VMEM((gsz, nq, _H), jnp.float32),
            pltpu.VMEM((gsz, nq, _H), jnp.float32),
        ],
        compiler_params=pltpu.CompilerParams(
            dimension_semantics=("parallel", "arbitrary", "arbitrary"),
            vmem_limit_bytes=48 * 1024 * 1024,
        ),
        name="spdecoder",
    )(inst5, query_embed,
      ca_wqkv, r3(ca_bqkv), ca_wo, r3(ca_bo),
      sa_wqkv, r3(sa_bqkv), sa_wo, r3(sa_bo), r3(sa_ln_g), r3(sa_ln_b),
      ffn_w1, r3(ffn_b1), ffn_w2, r3(ffn_b2), r3(ffn_ln_g), r3(ffn_ln_b),
      r2(out_ln_g), r2(out_ln_b), cls_w1, r2(cls_b1), cls_w2, r2(cls_b2))
    qn_res = qn_res.reshape(b, nq, d)
    labels = labels.reshape(b, nq, nc1)

    masks_t = pl.pallas_call(
        _mask_body,
        grid=(b, n_chunks),
        in_specs=[
            pl.BlockSpec((d, d_in), lambda bi, ci: (0, 0)),
            pl.BlockSpec((1, d), lambda bi, ci: (0, 0)),
            pl.BlockSpec((d, d), lambda bi, ci: (0, 0)),
            pl.BlockSpec((1, d), lambda bi, ci: (0, 0)),
            pl.BlockSpec((1, nq, d), lambda bi, ci: (bi, 0, 0)),
            pl.BlockSpec((1, cn, d_in), lambda bi, ci: (bi, ci, 0)),
        ],
        out_specs=pl.BlockSpec((1, cn, nq), lambda bi, ci: (bi, ci, 0)),
        out_shape=jax.ShapeDtypeStruct((b, n, nq), jnp.float32),
        compiler_params=pltpu.CompilerParams(
            dimension_semantics=("parallel", "arbitrary"),
            vmem_limit_bytes=48 * 1024 * 1024,
        ),
        name="spmasks",
    )(xm_w1, r2(xm_b1), xm_w2, r2(xm_b2), qn_res, x4)

    return labels, masks_t.transpose(0, 2, 1)
